# baseline (speedup 1.0000x reference)
import jax
import jax.numpy as jnp
from jax import lax
from jax.experimental import pallas as pl
from jax.experimental.pallas import tpu as pltpu
from jax.experimental.pallas import tpu_sc as plsc

N_ROWS = 320000
D = 128
NSEG = 10000
NW = 32
SPW = 313
SEG_PAD = NW * SPW
CHUNK = 128
CSHIFT = 7
NBUF = 4
NB = 48


def _splat_lane(vec, lane):
    idx = jnp.full((16, 1), lane, jnp.int32)
    dn = lax.GatherDimensionNumbers(
        offset_dims=(), collapsed_slice_dims=(0,), start_index_map=(0,))
    return lax.gather(vec, idx, dn, (1,),
                      mode=lax.GatherScatterMode.PROMISE_IN_BOUNDS)


def _sc_body(bounds_hbm, seq_hbm, ids_hbm, out_hbm,
             bounds_v, ids0, ids1, ids2, ids3, in0, in1, in2, in3,
             acc_v, cnt_v, sem0, sem1, sem2, sem3):
    wid = lax.axis_index("s") * 2 + lax.axis_index("c")

    pltpu.sync_copy(bounds_hbm, bounds_v)

    zeros = jnp.zeros((16,), jnp.float32)
    ones = jnp.ones((16,), jnp.float32)
    iota = lax.iota(jnp.int32, 16)

    seg_base = wid * SPW
    bv = bounds_v[pl.ds(wid, 16)]
    r0 = bv[0]
    r1 = bv[1]
    a0 = (r0 >> 3) << 3
    nchunks = (r1 - a0 + (CHUNK - 1)) >> CSHIFT

    ids_bufs = (ids0, ids1, ids2, ids3)
    in_bufs = (in0, in1, in2, in3)
    sems = (sem0, sem1, sem2, sem3)

    def chunk_start(j):
        return pl.multiple_of(jnp.minimum(a0 + j * CHUNK, N_ROWS - CHUNK), 8)

    def start_dmas(j, b):
        st = chunk_start(j)
        pltpu.async_copy(ids_hbm.at[pl.ds(st, CHUNK)], ids_bufs[b], sems[b])
        pltpu.async_copy(seq_hbm.at[pl.ds(st * D, CHUNK * D)], in_bufs[b], sems[b])

    def wait_dmas(b):
        pltpu.make_async_copy(ids_hbm.at[pl.ds(0, CHUNK)], ids_bufs[b], sems[b]).wait()
        pltpu.make_async_copy(seq_hbm.at[pl.ds(0, CHUNK * D)], in_bufs[b], sems[b]).wait()

    def process(j, b):
        ids_v = ids_bufs[b]
        in_v = in_bufs[b]
        st = chunk_start(j)
        lo = jnp.maximum(r0, a0 + j * CHUNK) - st
        hi = jnp.minimum(r1, st + CHUNK) - st

        def group_body(g, _):
            idv = ids_v[pl.ds(g * 16, 16)]
            full = (g * 16 >= lo) & (g * 16 + 16 <= hi)
            first = idv[0]
            last = idv[15]
            fsplat = _splat_lane(idv, 0)
            lsplat = _splat_lane(idv, 15)
            mask_a = idv == fsplat
            n_ok = plsc.all_reduce_population_count(mask_a | (idv == lsplat))
            fast = full & (n_ok[0] == 16)
            def fast_path():
                ma_i = mask_a.astype(jnp.int32)
                tot = [jnp.zeros((16,), jnp.float32) for _ in range(D // 16)]
                asum = [jnp.zeros((16,), jnp.float32) for _ in range(D // 16)]
                for lane in range(16):
                    bit = _splat_lane(ma_i, lane) != 0
                    rbase = (g * 16 + lane) * D
                    for k in range(D // 16):
                        row = in_v[pl.ds(rbase + k * 16, 16)]
                        tot[k] = tot[k] + row
                        asum[k] = asum[k] + jnp.where(bit, row, 0.0)
                off_a = (first - seg_base) * D
                off_b = (last - seg_base) * D
                for k in range(D // 16):
                    plsc.addupdate(acc_v.at[pl.ds(off_a + k * 16, 16)], asum[k])
                    plsc.addupdate(acc_v.at[pl.ds(off_b + k * 16, 16)],
                                   tot[k] - asum[k])
                cnt_a = plsc.all_reduce_population_count(mask_a).astype(jnp.float32)
                plsc.addupdate(cnt_v.at[pl.ds((first - seg_base) * 16, 16)], cnt_a)
                plsc.addupdate(cnt_v.at[pl.ds((last - seg_base) * 16, 16)],
                               16.0 - cnt_a)

            def slow_path():
                rowi = g * 16 + iota
                inr = (rowi >= lo) & (rowi < hi)
                locv = idv - seg_base
                offv = jnp.where(inr, locv * D, SPW * D)
                cofv = jnp.where(inr, locv * 16, SPW * 16)
                offs = [offv[lane] for lane in range(16)]
                cofs = [cofv[lane] for lane in range(16)]
                for lane in range(16):
                    rbase = (g * 16 + lane) * D
                    for k in range(D // 16):
                        plsc.addupdate(acc_v.at[pl.ds(offs[lane] + k * 16, 16)],
                                       in_v[pl.ds(rbase + k * 16, 16)])
                    plsc.addupdate(cnt_v.at[pl.ds(cofs[lane], 16)], ones)

            lax.cond(fast, fast_path, slow_path)
            return 0
        lax.fori_loop(lo >> 4, (hi + 15) >> 4, group_body, 0)

    for b in range(NBUF):
        @pl.when(b < nchunks)
        def _(b=b):
            start_dmas(b, b)

    def zero_body(s, _):
        for k in range(D // 16):
            acc_v[pl.ds(s * D + k * 16, 16)] = zeros
        cnt_v[pl.ds(s * 16, 16)] = zeros
        return 0
    lax.fori_loop(0, SPW, zero_body, 0)

    def ring_body(p, _):
        for b in range(NBUF):
            j = p * NBUF + b

            @pl.when(j < nchunks)
            def _(j=j, b=b):
                wait_dmas(b)
                process(j, b)

                @pl.when(j + NBUF < nchunks)
                def _(j=j, b=b):
                    start_dmas(j + NBUF, b)
        return 0
    lax.fori_loop(0, (nchunks + NBUF - 1) >> 2, ring_body, 0)

    def div_body(s, _):
        cv = cnt_v[pl.ds(s * 16, 16)]
        scale = 1.0 / (cv + 1.0)
        for k in range(D // 16):
            acc_v[pl.ds(s * D + k * 16, 16)] *= scale
        return 0
    lax.fori_loop(0, SPW, div_body, 0)

    pltpu.sync_copy(acc_v.at[pl.ds(0, SPW * D)],
                    out_hbm.at[pl.ds(seg_base * D, SPW * D)])


@jax.jit
def _sc_call(bounds, seqf, ids):
    mesh = plsc.VectorSubcoreMesh(core_axis_name="c", subcore_axis_name="s")
    return pl.kernel(
        _sc_body,
        mesh=mesh,
        compiler_params=pltpu.CompilerParams(needs_layout_passes=False),
        out_type=jax.ShapeDtypeStruct((SEG_PAD * D,), jnp.float32),
        scratch_types=[
            pltpu.VMEM((NB,), jnp.int32),
            pltpu.VMEM((CHUNK,), jnp.int32),
            pltpu.VMEM((CHUNK,), jnp.int32),
            pltpu.VMEM((CHUNK,), jnp.int32),
            pltpu.VMEM((CHUNK,), jnp.int32),
            pltpu.VMEM((CHUNK * D,), jnp.float32),
            pltpu.VMEM((CHUNK * D,), jnp.float32),
            pltpu.VMEM((CHUNK * D,), jnp.float32),
            pltpu.VMEM((CHUNK * D,), jnp.float32),
            pltpu.VMEM(((SPW + 1) * D,), jnp.float32),
            pltpu.VMEM(((SPW + 1) * 16,), jnp.float32),
            pltpu.SemaphoreType.DMA,
            pltpu.SemaphoreType.DMA,
            pltpu.SemaphoreType.DMA,
            pltpu.SemaphoreType.DMA,
        ],
    )(bounds, seqf, ids)


def kernel(seq, sub_match):
    ids = sub_match.astype(jnp.int32)
    marks = jnp.arange(NW + 1, dtype=jnp.int32) * SPW
    bounds = jnp.searchsorted(ids, marks).astype(jnp.int32)
    bounds = jnp.pad(bounds, (0, NB - (NW + 1)))
    out = _sc_call(bounds, seq.reshape(-1), ids)
    return out.reshape(SEG_PAD, D)[:NSEG]


# speedup vs baseline: 1.0345x; 1.0345x over previous
import jax
import jax.numpy as jnp
from jax import lax
from jax.experimental import pallas as pl
from jax.experimental.pallas import tpu as pltpu
from jax.experimental.pallas import tpu_sc as plsc

N_ROWS = 320000
D = 128
NSEG = 10000
NW = 32
SPW = 313
SEG_PAD = NW * SPW
CHUNK = 256
CSHIFT = 8
NBUF = 2
NB = 48


def _splat_lane(vec, lane):
    idx = jnp.full((16, 1), lane, jnp.int32)
    dn = lax.GatherDimensionNumbers(
        offset_dims=(), collapsed_slice_dims=(0,), start_index_map=(0,))
    return lax.gather(vec, idx, dn, (1,),
                      mode=lax.GatherScatterMode.PROMISE_IN_BOUNDS)


def _sc_body(bounds_hbm, seq_hbm, ids_hbm, out_hbm,
             bounds_v, ids0, ids1, in0, in1,
             acc_v, cnt_v, sem0, sem1):
    wid = lax.axis_index("s") * 2 + lax.axis_index("c")

    pltpu.sync_copy(bounds_hbm, bounds_v)

    zeros = jnp.zeros((16,), jnp.float32)
    ones = jnp.ones((16,), jnp.float32)
    iota = lax.iota(jnp.int32, 16)

    seg_base = wid * SPW
    bv = bounds_v[pl.ds(wid, 16)]
    r0 = bv[0]
    r1 = bv[1]
    a0 = (r0 >> 3) << 3
    nchunks = (r1 - a0 + (CHUNK - 1)) >> CSHIFT

    ids_bufs = (ids0, ids1)
    in_bufs = (in0, in1)
    sems = (sem0, sem1)

    def chunk_start(j):
        return pl.multiple_of(jnp.minimum(a0 + j * CHUNK, N_ROWS - CHUNK), 8)

    def start_dmas(j, b):
        st = chunk_start(j)
        pltpu.async_copy(ids_hbm.at[pl.ds(st, CHUNK)], ids_bufs[b], sems[b])
        pltpu.async_copy(seq_hbm.at[pl.ds(st * D, CHUNK * D)], in_bufs[b], sems[b])

    def wait_dmas(b):
        pltpu.make_async_copy(ids_hbm.at[pl.ds(0, CHUNK)], ids_bufs[b], sems[b]).wait()
        pltpu.make_async_copy(seq_hbm.at[pl.ds(0, CHUNK * D)], in_bufs[b], sems[b]).wait()

    def process(j, b):
        ids_v = ids_bufs[b]
        in_v = in_bufs[b]
        st = chunk_start(j)
        lo = jnp.maximum(r0, a0 + j * CHUNK) - st
        hi = jnp.minimum(r1, st + CHUNK) - st

        def group_body(g, _):
            idv = ids_v[pl.ds(g * 16, 16)]
            full = (g * 16 >= lo) & (g * 16 + 16 <= hi)
            first = idv[0]
            last = idv[15]
            fsplat = _splat_lane(idv, 0)
            lsplat = _splat_lane(idv, 15)
            mask_a = idv == fsplat
            n_ok = plsc.all_reduce_population_count(mask_a | (idv == lsplat))
            fast = full & (n_ok[0] == 16)
            def fast_path():
                ma_i = mask_a.astype(jnp.int32)
                tot = [jnp.zeros((16,), jnp.float32) for _ in range(D // 16)]
                asum = [jnp.zeros((16,), jnp.float32) for _ in range(D // 16)]
                for lane in range(16):
                    bit = _splat_lane(ma_i, lane) != 0
                    rbase = (g * 16 + lane) * D
                    for k in range(D // 16):
                        row = in_v[pl.ds(rbase + k * 16, 16)]
                        tot[k] = tot[k] + row
                        asum[k] = asum[k] + jnp.where(bit, row, 0.0)
                off_a = (first - seg_base) * D
                off_b = (last - seg_base) * D
                for k in range(D // 16):
                    plsc.addupdate(acc_v.at[pl.ds(off_a + k * 16, 16)], asum[k])
                    plsc.addupdate(acc_v.at[pl.ds(off_b + k * 16, 16)],
                                   tot[k] - asum[k])
                cnt_a = plsc.all_reduce_population_count(mask_a).astype(jnp.float32)
                plsc.addupdate(cnt_v.at[pl.ds((first - seg_base) * 16, 16)], cnt_a)
                plsc.addupdate(cnt_v.at[pl.ds((last - seg_base) * 16, 16)],
                               16.0 - cnt_a)

            def slow_path():
                rowi = g * 16 + iota
                inr = (rowi >= lo) & (rowi < hi)
                locv = idv - seg_base
                offv = jnp.where(inr, locv * D, SPW * D)
                cofv = jnp.where(inr, locv * 16, SPW * 16)
                offs = [offv[lane] for lane in range(16)]
                cofs = [cofv[lane] for lane in range(16)]
                for lane in range(16):
                    rbase = (g * 16 + lane) * D
                    for k in range(D // 16):
                        plsc.addupdate(acc_v.at[pl.ds(offs[lane] + k * 16, 16)],
                                       in_v[pl.ds(rbase + k * 16, 16)])
                    plsc.addupdate(cnt_v.at[pl.ds(cofs[lane], 16)], ones)

            lax.cond(fast, fast_path, slow_path)
            return 0
        lax.fori_loop(lo >> 4, (hi + 15) >> 4, group_body, 0)

    for b in range(NBUF):
        @pl.when(b < nchunks)
        def _(b=b):
            start_dmas(b, b)

    def zero_body(s, _):
        for k in range(D // 16):
            acc_v[pl.ds(s * D + k * 16, 16)] = zeros
        cnt_v[pl.ds(s * 16, 16)] = zeros
        return 0
    lax.fori_loop(0, SPW, zero_body, 0)

    def ring_body(p, _):
        for b in range(NBUF):
            j = p * NBUF + b

            @pl.when(j < nchunks)
            def _(j=j, b=b):
                wait_dmas(b)
                process(j, b)

                @pl.when(j + NBUF < nchunks)
                def _(j=j, b=b):
                    start_dmas(j + NBUF, b)
        return 0
    lax.fori_loop(0, (nchunks + NBUF - 1) >> 1, ring_body, 0)

    def div_body(s, _):
        cv = cnt_v[pl.ds(s * 16, 16)]
        scale = 1.0 / (cv + 1.0)
        for k in range(D // 16):
            acc_v[pl.ds(s * D + k * 16, 16)] *= scale
        return 0
    lax.fori_loop(0, SPW, div_body, 0)

    pltpu.sync_copy(acc_v.at[pl.ds(0, SPW * D)],
                    out_hbm.at[pl.ds(seg_base * D, SPW * D)])


@jax.jit
def _sc_call(bounds, seqf, ids):
    mesh = plsc.VectorSubcoreMesh(core_axis_name="c", subcore_axis_name="s")
    return pl.kernel(
        _sc_body,
        mesh=mesh,
        compiler_params=pltpu.CompilerParams(needs_layout_passes=False),
        out_type=jax.ShapeDtypeStruct((SEG_PAD * D,), jnp.float32),
        scratch_types=[
            pltpu.VMEM((NB,), jnp.int32),
            pltpu.VMEM((CHUNK,), jnp.int32),
            pltpu.VMEM((CHUNK,), jnp.int32),
            pltpu.VMEM((CHUNK * D,), jnp.float32),
            pltpu.VMEM((CHUNK * D,), jnp.float32),
            pltpu.VMEM(((SPW + 1) * D,), jnp.float32),
            pltpu.VMEM(((SPW + 1) * 16,), jnp.float32),
            pltpu.SemaphoreType.DMA,
            pltpu.SemaphoreType.DMA,
        ],
    )(bounds, seqf, ids)


def kernel(seq, sub_match):
    ids = sub_match.astype(jnp.int32)
    marks = jnp.arange(NW + 1, dtype=jnp.int32) * SPW
    bounds = jnp.searchsorted(ids, marks).astype(jnp.int32)
    bounds = jnp.pad(bounds, (0, NB - (NW + 1)))
    out = _sc_call(bounds, seq.reshape(-1), ids)
    return out.reshape(SEG_PAD, D)[:NSEG]


# speedup vs baseline: 1.0417x; 1.0069x over previous
import jax
import jax.numpy as jnp
from jax import lax
from jax.experimental import pallas as pl
from jax.experimental.pallas import tpu as pltpu
from jax.experimental.pallas import tpu_sc as plsc

N_ROWS = 320000
D = 128
NSEG = 10000
NW = 32
SPW = 313
SEG_PAD = NW * SPW
CHUNK = 320
CSHIFT = 0
NBUF = 2
NB = 48


def _splat_lane(vec, lane):
    idx = jnp.full((16, 1), lane, jnp.int32)
    dn = lax.GatherDimensionNumbers(
        offset_dims=(), collapsed_slice_dims=(0,), start_index_map=(0,))
    return lax.gather(vec, idx, dn, (1,),
                      mode=lax.GatherScatterMode.PROMISE_IN_BOUNDS)


def _sc_body(bounds_hbm, seq_hbm, ids_hbm, out_hbm,
             bounds_v, ids0, ids1, in0, in1,
             acc_v, cnt_v, sem0, sem1):
    wid = lax.axis_index("s") * 2 + lax.axis_index("c")

    pltpu.sync_copy(bounds_hbm, bounds_v)

    zeros = jnp.zeros((16,), jnp.float32)
    ones = jnp.ones((16,), jnp.float32)
    iota = lax.iota(jnp.int32, 16)

    seg_base = wid * SPW
    bv = bounds_v[pl.ds(wid, 16)]
    r0 = bv[0]
    r1 = bv[1]
    a0 = (r0 >> 3) << 3
    nchunks = lax.div(r1 - a0 + (CHUNK - 1), CHUNK)

    ids_bufs = (ids0, ids1)
    in_bufs = (in0, in1)
    sems = (sem0, sem1)

    def chunk_start(j):
        return pl.multiple_of(jnp.minimum(a0 + j * CHUNK, N_ROWS - CHUNK), 8)

    def start_dmas(j, b):
        st = chunk_start(j)
        pltpu.async_copy(ids_hbm.at[pl.ds(st, CHUNK)], ids_bufs[b], sems[b])
        pltpu.async_copy(seq_hbm.at[pl.ds(st * D, CHUNK * D)], in_bufs[b], sems[b])

    def wait_dmas(b):
        pltpu.make_async_copy(ids_hbm.at[pl.ds(0, CHUNK)], ids_bufs[b], sems[b]).wait()
        pltpu.make_async_copy(seq_hbm.at[pl.ds(0, CHUNK * D)], in_bufs[b], sems[b]).wait()

    def process(j, b):
        ids_v = ids_bufs[b]
        in_v = in_bufs[b]
        st = chunk_start(j)
        lo = jnp.maximum(r0, a0 + j * CHUNK) - st
        hi = jnp.minimum(r1, st + CHUNK) - st

        def group_body(g, _):
            idv = ids_v[pl.ds(g * 16, 16)]
            full = (g * 16 >= lo) & (g * 16 + 16 <= hi)
            first = idv[0]
            last = idv[15]
            fsplat = _splat_lane(idv, 0)
            lsplat = _splat_lane(idv, 15)
            mask_a = idv == fsplat
            n_ok = plsc.all_reduce_population_count(mask_a | (idv == lsplat))
            fast = full & (n_ok[0] == 16)
            def fast_path():
                ma_i = mask_a.astype(jnp.int32)
                tot = [jnp.zeros((16,), jnp.float32) for _ in range(D // 16)]
                asum = [jnp.zeros((16,), jnp.float32) for _ in range(D // 16)]
                for lane in range(16):
                    bit = _splat_lane(ma_i, lane) != 0
                    rbase = (g * 16 + lane) * D
                    for k in range(D // 16):
                        row = in_v[pl.ds(rbase + k * 16, 16)]
                        tot[k] = tot[k] + row
                        asum[k] = asum[k] + jnp.where(bit, row, 0.0)
                off_a = (first - seg_base) * D
                off_b = (last - seg_base) * D
                for k in range(D // 16):
                    plsc.addupdate(acc_v.at[pl.ds(off_a + k * 16, 16)], asum[k])
                    plsc.addupdate(acc_v.at[pl.ds(off_b + k * 16, 16)],
                                   tot[k] - asum[k])
                cnt_a = plsc.all_reduce_population_count(mask_a).astype(jnp.float32)
                plsc.addupdate(cnt_v.at[pl.ds((first - seg_base) * 16, 16)], cnt_a)
                plsc.addupdate(cnt_v.at[pl.ds((last - seg_base) * 16, 16)],
                               16.0 - cnt_a)

            def slow_path():
                rowi = g * 16 + iota
                inr = (rowi >= lo) & (rowi < hi)
                locv = idv - seg_base
                offv = jnp.where(inr, locv * D, SPW * D)
                cofv = jnp.where(inr, locv * 16, SPW * 16)
                offs = [offv[lane] for lane in range(16)]
                cofs = [cofv[lane] for lane in range(16)]
                for lane in range(16):
                    rbase = (g * 16 + lane) * D
                    for k in range(D // 16):
                        plsc.addupdate(acc_v.at[pl.ds(offs[lane] + k * 16, 16)],
                                       in_v[pl.ds(rbase + k * 16, 16)])
                    plsc.addupdate(cnt_v.at[pl.ds(cofs[lane], 16)], ones)

            lax.cond(fast, fast_path, slow_path)
            return 0
        lax.fori_loop(lo >> 4, (hi + 15) >> 4, group_body, 0)

    for b in range(NBUF):
        @pl.when(b < nchunks)
        def _(b=b):
            start_dmas(b, b)

    def zero_body(s, _):
        for k in range(D // 16):
            acc_v[pl.ds(s * D + k * 16, 16)] = zeros
        cnt_v[pl.ds(s * 16, 16)] = zeros
        return 0
    lax.fori_loop(0, SPW, zero_body, 0)

    def ring_body(p, _):
        for b in range(NBUF):
            j = p * NBUF + b

            @pl.when(j < nchunks)
            def _(j=j, b=b):
                wait_dmas(b)
                process(j, b)

                @pl.when(j + NBUF < nchunks)
                def _(j=j, b=b):
                    start_dmas(j + NBUF, b)
        return 0
    lax.fori_loop(0, lax.div(nchunks + NBUF - 1, NBUF), ring_body, 0)

    def div_body(s, _):
        cv = cnt_v[pl.ds(s * 16, 16)]
        scale = 1.0 / (cv + 1.0)
        for k in range(D // 16):
            acc_v[pl.ds(s * D + k * 16, 16)] *= scale
        return 0
    lax.fori_loop(0, SPW, div_body, 0)

    pltpu.sync_copy(acc_v.at[pl.ds(0, SPW * D)],
                    out_hbm.at[pl.ds(seg_base * D, SPW * D)])


@jax.jit
def _sc_call(bounds, seqf, ids):
    mesh = plsc.VectorSubcoreMesh(core_axis_name="c", subcore_axis_name="s")
    return pl.kernel(
        _sc_body,
        mesh=mesh,
        compiler_params=pltpu.CompilerParams(needs_layout_passes=False),
        out_type=jax.ShapeDtypeStruct((SEG_PAD * D,), jnp.float32),
        scratch_types=[
            pltpu.VMEM((NB,), jnp.int32),
            pltpu.VMEM((CHUNK,), jnp.int32),
            pltpu.VMEM((CHUNK,), jnp.int32),
            pltpu.VMEM((CHUNK * D,), jnp.float32),
            pltpu.VMEM((CHUNK * D,), jnp.float32),
            pltpu.VMEM(((SPW + 1) * D,), jnp.float32),
            pltpu.VMEM(((SPW + 1) * 16,), jnp.float32),
            pltpu.SemaphoreType.DMA,
            pltpu.SemaphoreType.DMA,
        ],
    )(bounds, seqf, ids)


def kernel(seq, sub_match):
    ids = sub_match.astype(jnp.int32)
    marks = jnp.arange(NW + 1, dtype=jnp.int32) * SPW
    bounds = jnp.searchsorted(ids, marks).astype(jnp.int32)
    bounds = jnp.pad(bounds, (0, NB - (NW + 1)))
    out = _sc_call(bounds, seq.reshape(-1), ids)
    return out.reshape(SEG_PAD, D)[:NSEG]


# speedup vs baseline: 1.0493x; 1.0073x over previous
import jax
import jax.numpy as jnp
from jax import lax
from jax.experimental import pallas as pl
from jax.experimental.pallas import tpu as pltpu
from jax.experimental.pallas import tpu_sc as plsc

N_ROWS = 320000
D = 128
NSEG = 10000
NW = 32
SPW = 313
SEG_PAD = NW * SPW
CHUNK = 320
CSHIFT = 0
NBUF = 2
NB = 48


def _splat_lane(vec, lane):
    idx = jnp.full((16, 1), lane, jnp.int32)
    dn = lax.GatherDimensionNumbers(
        offset_dims=(), collapsed_slice_dims=(0,), start_index_map=(0,))
    return lax.gather(vec, idx, dn, (1,),
                      mode=lax.GatherScatterMode.PROMISE_IN_BOUNDS)


def _sc_body(bounds_hbm, seq_hbm, ids_hbm, out_hbm,
             bounds_v, ids0, ids1, in0, in1,
             acc_v, cnt_v, sem0, sem1):
    wid = lax.axis_index("s") * 2 + lax.axis_index("c")

    pltpu.sync_copy(bounds_hbm, bounds_v)

    zeros = jnp.zeros((16,), jnp.float32)
    ones = jnp.ones((16,), jnp.float32)
    iota = lax.iota(jnp.int32, 16)

    seg_base = wid * SPW
    bv = bounds_v[pl.ds(wid, 16)]
    r0 = bv[0]
    r1 = bv[1]
    a0 = (r0 >> 3) << 3
    nchunks = lax.div(r1 - a0 + (CHUNK - 1), CHUNK)

    ids_bufs = (ids0, ids1)
    in_bufs = (in0, in1)
    sems = (sem0, sem1)

    def chunk_start(j):
        return pl.multiple_of(jnp.minimum(a0 + j * CHUNK, N_ROWS - CHUNK), 8)

    def start_dmas(j, b):
        st = chunk_start(j)
        pltpu.async_copy(ids_hbm.at[pl.ds(st, CHUNK)], ids_bufs[b], sems[b])
        pltpu.async_copy(seq_hbm.at[pl.ds(st * D, CHUNK * D)], in_bufs[b], sems[b])

    def wait_dmas(b):
        pltpu.make_async_copy(ids_hbm.at[pl.ds(0, CHUNK)], ids_bufs[b], sems[b]).wait()
        pltpu.make_async_copy(seq_hbm.at[pl.ds(0, CHUNK * D)], in_bufs[b], sems[b]).wait()

    def process(j, b):
        ids_v = ids_bufs[b]
        in_v = in_bufs[b]
        st = chunk_start(j)
        lo = jnp.maximum(r0, a0 + j * CHUNK) - st
        hi = jnp.minimum(r1, st + CHUNK) - st

        def group_body(g, _):
            idv = ids_v[pl.ds(g * 16, 16)]
            full = (g * 16 >= lo) & (g * 16 + 16 <= hi)
            first = idv[0]
            last = idv[15]
            fsplat = _splat_lane(idv, 0)
            lsplat = _splat_lane(idv, 15)
            mask_a = idv == fsplat
            n_ok = plsc.all_reduce_population_count(mask_a | (idv == lsplat))
            fast = full & (n_ok[0] == 16)
            def fast_path():
                ma_i = mask_a.astype(jnp.int32)
                tot = [jnp.zeros((16,), jnp.float32) for _ in range(D // 16)]
                asum = [jnp.zeros((16,), jnp.float32) for _ in range(D // 16)]
                for lane in range(16):
                    bit = _splat_lane(ma_i, lane) != 0
                    rbase = (g * 16 + lane) * D
                    for k in range(D // 16):
                        row = in_v[pl.ds(rbase + k * 16, 16)]
                        tot[k] = tot[k] + row
                        asum[k] = asum[k] + jnp.where(bit, row, 0.0)
                off_a = (first - seg_base) * D
                off_b = (last - seg_base) * D
                for k in range(D // 16):
                    plsc.addupdate(acc_v.at[pl.ds(off_a + k * 16, 16)], asum[k])
                    plsc.addupdate(acc_v.at[pl.ds(off_b + k * 16, 16)],
                                   tot[k] - asum[k])
                cnt_a = plsc.all_reduce_population_count(mask_a).astype(jnp.float32)
                plsc.addupdate(cnt_v.at[pl.ds((first - seg_base) * 16, 16)], cnt_a)
                plsc.addupdate(cnt_v.at[pl.ds((last - seg_base) * 16, 16)],
                               16.0 - cnt_a)

            def slow_path():
                rowi = g * 16 + iota
                inr = (rowi >= lo) & (rowi < hi)
                locv = idv - seg_base
                offv = jnp.where(inr, locv * D, SPW * D)
                cofv = jnp.where(inr, locv * 16, SPW * 16)
                offs = [offv[lane] for lane in range(16)]
                cofs = [cofv[lane] for lane in range(16)]
                for lane in range(16):
                    rbase = (g * 16 + lane) * D
                    for k in range(D // 16):
                        plsc.addupdate(acc_v.at[pl.ds(offs[lane] + k * 16, 16)],
                                       in_v[pl.ds(rbase + k * 16, 16)])
                    plsc.addupdate(cnt_v.at[pl.ds(cofs[lane], 16)], ones)

            lax.cond(fast, fast_path, slow_path)
            return 0
        lax.fori_loop(lo >> 4, (hi + 15) >> 4, group_body, 0)

    for b in range(NBUF):
        @pl.when(b < nchunks)
        def _(b=b):
            start_dmas(b, b)

    def zero_body(s, _):
        for k in range(D // 16):
            acc_v[pl.ds(s * D + k * 16, 16)] = zeros
        cnt_v[pl.ds(s * 16, 16)] = zeros
        return 0
    lax.fori_loop(0, SPW, zero_body, 0)

    def ring_body(p, _):
        for b in range(NBUF):
            j = p * NBUF + b

            @pl.when(j < nchunks)
            def _(j=j, b=b):
                wait_dmas(b)
                process(j, b)

                @pl.when(j + NBUF < nchunks)
                def _(j=j, b=b):
                    start_dmas(j + NBUF, b)
        return 0
    lax.fori_loop(0, lax.div(nchunks + NBUF - 1, NBUF), ring_body, 0)

    cuts = (0, 79, 158, 237, SPW)

    def div_body(s, _):
        cv = cnt_v[pl.ds(s * 16, 16)]
        scale = 1.0 / (cv + 1.0)
        for k in range(D // 16):
            acc_v[pl.ds(s * D + k * 16, 16)] *= scale
        return 0

    for t in range(4):
        lax.fori_loop(cuts[t], cuts[t + 1], div_body, 0)
        n = (cuts[t + 1] - cuts[t]) * D
        pltpu.async_copy(acc_v.at[pl.ds(cuts[t] * D, n)],
                         out_hbm.at[pl.ds(seg_base * D + cuts[t] * D, n)],
                         sems[0])
    for t in range(4):
        n = (cuts[t + 1] - cuts[t]) * D
        pltpu.make_async_copy(acc_v.at[pl.ds(cuts[t] * D, n)],
                              out_hbm.at[pl.ds(seg_base * D + cuts[t] * D, n)],
                              sems[0]).wait()


@jax.jit
def _sc_call(bounds, seqf, ids):
    mesh = plsc.VectorSubcoreMesh(core_axis_name="c", subcore_axis_name="s")
    return pl.kernel(
        _sc_body,
        mesh=mesh,
        compiler_params=pltpu.CompilerParams(needs_layout_passes=False),
        out_type=jax.ShapeDtypeStruct((SEG_PAD * D,), jnp.float32),
        scratch_types=[
            pltpu.VMEM((NB,), jnp.int32),
            pltpu.VMEM((CHUNK,), jnp.int32),
            pltpu.VMEM((CHUNK,), jnp.int32),
            pltpu.VMEM((CHUNK * D,), jnp.float32),
            pltpu.VMEM((CHUNK * D,), jnp.float32),
            pltpu.VMEM(((SPW + 1) * D,), jnp.float32),
            pltpu.VMEM(((SPW + 1) * 16,), jnp.float32),
            pltpu.SemaphoreType.DMA,
            pltpu.SemaphoreType.DMA,
        ],
    )(bounds, seqf, ids)


def kernel(seq, sub_match):
    ids = sub_match.astype(jnp.int32)
    marks = jnp.arange(NW + 1, dtype=jnp.int32) * SPW
    bounds = jnp.searchsorted(ids, marks).astype(jnp.int32)
    bounds = jnp.pad(bounds, (0, NB - (NW + 1)))
    out = _sc_call(bounds, seq.reshape(-1), ids)
    return out.reshape(SEG_PAD, D)[:NSEG]
